# baseline (speedup 1.0000x reference)
import functools

import jax
import jax.numpy as jnp
from jax.experimental import pallas as pl

_TILE = 256
_N = 32768
_K = 1024
_D = 64
_L = 4
_BETA = 0.25


def _rvq_body(x_ref, cb_ref, cbh_ref, cbm_ref, cbl_ref, xq_ref, idx_ref,
              loss_ref):
    r = x_ref[...]
    total = jnp.zeros((), jnp.float32)
    lane = jax.lax.broadcasted_iota(jnp.int32, (r.shape[0], _K), 1)
    for i in range(_L):
        cb = cb_ref[i]
        c2 = jnp.sum(cb * cb, axis=1)[None, :]
        r2 = jnp.sum(r * r, axis=1, keepdims=True)
        mm = jnp.dot(r, cbh_ref[i])
        mm += jnp.dot(r, cbm_ref[i])
        mm += jnp.dot(r, cbl_ref[i])
        d = (r2 + c2) - 2.0 * mm
        dmin = jnp.min(d, axis=1, keepdims=True)
        idx = jnp.min(jnp.where(d == dmin, lane, _K), axis=1, keepdims=True)
        oh = (lane == idx).astype(jnp.float32)
        q = jnp.dot(oh, cb, preferred_element_type=jnp.float32,
                    precision=jax.lax.Precision.HIGHEST)
        e = q - r
        total = total + jnp.sum(e * e)
        idx_ref[:, i] = idx[:, 0]
        r = r - q
    xq_ref[...] = x_ref[...] - r

    @pl.when(pl.program_id(0) == 0)
    def _():
        loss_ref[...] = total.reshape(1, 1)

    @pl.when(pl.program_id(0) != 0)
    def _():
        loss_ref[...] += total.reshape(1, 1)


@functools.partial(jax.jit, static_argnames=())
def kernel(x, codebooks):
    n_tiles = _N // _TILE
    cbt = jnp.swapaxes(codebooks, 1, 2)
    cbh = cbt.astype(jnp.bfloat16).astype(jnp.float32)
    rem = cbt - cbh
    cbm = rem.astype(jnp.bfloat16).astype(jnp.float32)
    cbl = (rem - cbm).astype(jnp.bfloat16).astype(jnp.float32)
    full_cbt = pl.BlockSpec((_L, _D, _K), lambda t: (0, 0, 0))
    xq, idx, loss_sum = pl.pallas_call(
        _rvq_body,
        grid=(n_tiles,),
        in_specs=[
            pl.BlockSpec((_TILE, _D), lambda t: (t, 0)),
            pl.BlockSpec((_L, _K, _D), lambda t: (0, 0, 0)),
            full_cbt,
            full_cbt,
            full_cbt,
        ],
        out_specs=[
            pl.BlockSpec((_TILE, _D), lambda t: (t, 0)),
            pl.BlockSpec((_TILE, _L), lambda t: (t, 0)),
            pl.BlockSpec((1, 1), lambda t: (0, 0)),
        ],
        out_shape=[
            jax.ShapeDtypeStruct((_N, _D), jnp.float32),
            jax.ShapeDtypeStruct((_N, _L), jnp.int32),
            jax.ShapeDtypeStruct((1, 1), jnp.float32),
        ],
    )(x, codebooks, cbh, cbm, cbl)
    mean_loss = (1.0 + _BETA) * loss_sum[0, 0] / (_L * _N * _D)
    return (xq, mean_loss, idx)


# speedup vs baseline: 1.7771x; 1.7771x over previous
import functools

import jax
import jax.numpy as jnp
from jax.experimental import pallas as pl

_TILE = 1024
_HALF = _TILE // 2
_N = 32768
_K = 1024
_D = 64
_L = 4
_BETA = 0.25


def _rvq_body(x_ref, cb_ref, cbh_ref, cbm_ref, cbl_ref,
              xq_ref, idx_ref, loss_ref):
    ra = x_ref[:_HALF]
    rb = x_ref[_HALF:]
    total = jnp.zeros((), jnp.float32)
    lane = jax.lax.broadcasted_iota(jnp.int32, (_HALF, _K), 1)

    def level(r, i, c2):
        r2 = jnp.sum(r * r, axis=1, keepdims=True)
        mm = jnp.dot(r, cbh_ref[i])
        mm += jnp.dot(r, cbm_ref[i])
        mm += jnp.dot(r, cbl_ref[i])
        d = (r2 + c2) - 2.0 * mm
        dmin = jnp.min(d, axis=1, keepdims=True)
        idx = jnp.min(jnp.where(d == dmin, lane, _K), axis=1, keepdims=True)
        oh = (lane == idx).astype(jnp.float32)
        q = jnp.dot(oh, cb_ref[i], preferred_element_type=jnp.float32,
                    precision=jax.lax.Precision.HIGHEST)
        e = q - r
        return r - q, idx, jnp.sum(e * e)

    for i in range(_L):
        cb = cb_ref[i]
        c2 = jnp.sum(cb * cb, axis=1)[None, :]
        ra, idxa, ta = level(ra, i, c2)
        rb, idxb, tb = level(rb, i, c2)
        total = total + (ta + tb)
        idx_ref[:_HALF, i] = idxa[:, 0]
        idx_ref[_HALF:, i] = idxb[:, 0]
    xq_ref[:_HALF] = x_ref[:_HALF] - ra
    xq_ref[_HALF:] = x_ref[_HALF:] - rb

    @pl.when(pl.program_id(0) == 0)
    def _():
        loss_ref[...] = total.reshape(1, 1)

    @pl.when(pl.program_id(0) != 0)
    def _():
        loss_ref[...] += total.reshape(1, 1)


@functools.partial(jax.jit, static_argnames=())
def kernel(x, codebooks):
    n_tiles = _N // _TILE
    ch32 = codebooks.astype(jnp.bfloat16).astype(jnp.float32)
    rem = codebooks - ch32
    cm32 = rem.astype(jnp.bfloat16).astype(jnp.float32)
    cl32 = (rem - cm32).astype(jnp.bfloat16).astype(jnp.float32)
    full_cb = pl.BlockSpec((_L, _K, _D), lambda t: (0, 0, 0))
    full_cbt = pl.BlockSpec((_L, _D, _K), lambda t: (0, 0, 0))
    xq, idx, loss_sum = pl.pallas_call(
        _rvq_body,
        grid=(n_tiles,),
        in_specs=[
            pl.BlockSpec((_TILE, _D), lambda t: (t, 0)),
            full_cb,
            full_cbt,
            full_cbt,
            full_cbt,
        ],
        out_specs=[
            pl.BlockSpec((_TILE, _D), lambda t: (t, 0)),
            pl.BlockSpec((_TILE, _L), lambda t: (t, 0)),
            pl.BlockSpec((1, 1), lambda t: (0, 0)),
        ],
        out_shape=[
            jax.ShapeDtypeStruct((_N, _D), jnp.float32),
            jax.ShapeDtypeStruct((_N, _L), jnp.int32),
            jax.ShapeDtypeStruct((1, 1), jnp.float32),
        ],
    )(x, codebooks, jnp.swapaxes(ch32, 1, 2), jnp.swapaxes(cm32, 1, 2),
      jnp.swapaxes(cl32, 1, 2))
    mean_loss = (1.0 + _BETA) * loss_sum[0, 0] / (_L * _N * _D)
    return (xq, mean_loss, idx)
